# baseline (speedup 1.0000x reference)
import jax
import jax.numpy as jnp
from jax import lax
from jax.experimental import pallas as pl
from jax.experimental.pallas import tpu as pltpu
from jax.experimental.pallas import tpu_sc as plsc

_B = 4096
_NC = 2
_NS = 16
_NW = _NC * _NS
_BPW = _B // _NW
_L = 16


def _body(idx_hbm, betas_hbm, comb_hbm,
          out_b, out_comb,
          idx_v, b_row, comb_v, sem):
    wid = lax.axis_index("s") * _NC + lax.axis_index("c")
    base = wid * _BPW

    pltpu.sync_copy(idx_hbm.at[pl.ds(base, _BPW)], idx_v)
    pltpu.sync_copy(betas_hbm, b_row)

    pending = []
    for r in range(_BPW):
        pending.append(
            pltpu.async_copy(b_row, out_b.at[pl.ds(base + r, 1)], sem))
    for g in range(_BPW // _L):
        v = idx_v[pl.ds(g * _L, _L)]
        for l in range(_L):
            r = g * _L + l
            s = v[l]
            pending.append(pltpu.async_copy(
                comb_hbm.at[pl.ds(s, 1)], comb_v.at[pl.ds(r, 1)], sem))
    for h in pending:
        h.wait()

    pltpu.sync_copy(comb_v, out_comb.at[pl.ds(base, _BPW)])


def kernel(idx, betas, global_orient, body_pose, transl):
    comb = jnp.concatenate([global_orient, transl, body_pose], axis=1)
    mesh = plsc.VectorSubcoreMesh(core_axis_name="c", subcore_axis_name="s")
    k = pl.kernel(
        _body,
        mesh=mesh,
        out_type=(
            jax.ShapeDtypeStruct((_B, 10), jnp.float32),
            jax.ShapeDtypeStruct((_B, 75), jnp.float32),
        ),
        scratch_types=[
            pltpu.VMEM((_BPW,), jnp.int32),
            pltpu.VMEM((1, 10), jnp.float32),
            pltpu.VMEM((_BPW, 75), jnp.float32),
            pltpu.SemaphoreType.DMA,
        ],
    )
    b, oc = k(idx.astype(jnp.int32), betas, comb)
    return (b, oc[:, 0:3], oc[:, 6:75], oc[:, 3:6])


# speedup vs baseline: 1.5627x; 1.5627x over previous
import jax
import jax.numpy as jnp
from jax import lax
from jax.experimental import pallas as pl
from jax.experimental.pallas import tpu as pltpu
from jax.experimental.pallas import tpu_sc as plsc

_B = 4096
_NC = 2
_NS = 16
_NW = _NC * _NS
_BPW = _B // _NW
_L = 16


def _body(idx_hbm, betas_hbm, go_hbm, bp_hbm, tr_hbm,
          out_b, out_go, out_bp, out_tr,
          idx_v, b_blk, go_v, bp_v, tr_v, sem):
    wid = lax.axis_index("s") * _NC + lax.axis_index("c")
    base = wid * _BPW

    pltpu.sync_copy(idx_hbm.at[pl.ds(base, _BPW)], idx_v)

    pending = []
    for l in range(_L):
        pending.append(pltpu.async_copy(betas_hbm, b_blk.at[pl.ds(l, 1)], sem))
    for h in pending:
        h.wait()
    pending = []
    for j in range(_BPW // _L):
        pending.append(
            pltpu.async_copy(b_blk, out_b.at[pl.ds(base + j * _L, _L)], sem))

    for g in range(_BPW // _L):
        v = idx_v[pl.ds(g * _L, _L)]
        for l in range(_L):
            r = g * _L + l
            s = v[l]
            pending.append(
                pltpu.async_copy(go_hbm.at[pl.ds(s, 1)], go_v.at[pl.ds(r, 1)], sem))
            pending.append(
                pltpu.async_copy(bp_hbm.at[pl.ds(s, 1)], bp_v.at[pl.ds(r, 1)], sem))
            pending.append(
                pltpu.async_copy(tr_hbm.at[pl.ds(s, 1)], tr_v.at[pl.ds(r, 1)], sem))
    for h in pending:
        h.wait()

    pltpu.sync_copy(go_v, out_go.at[pl.ds(base, _BPW)])
    pltpu.sync_copy(bp_v, out_bp.at[pl.ds(base, _BPW)])
    pltpu.sync_copy(tr_v, out_tr.at[pl.ds(base, _BPW)])


def kernel(idx, betas, global_orient, body_pose, transl):
    mesh = plsc.VectorSubcoreMesh(core_axis_name="c", subcore_axis_name="s")
    k = pl.kernel(
        _body,
        mesh=mesh,
        out_type=(
            jax.ShapeDtypeStruct((_B, 10), jnp.float32),
            jax.ShapeDtypeStruct((_B, 3), jnp.float32),
            jax.ShapeDtypeStruct((_B, 69), jnp.float32),
            jax.ShapeDtypeStruct((_B, 3), jnp.float32),
        ),
        scratch_types=[
            pltpu.VMEM((_BPW,), jnp.int32),
            pltpu.VMEM((_L, 10), jnp.float32),
            pltpu.VMEM((_BPW, 3), jnp.float32),
            pltpu.VMEM((_BPW, 69), jnp.float32),
            pltpu.VMEM((_BPW, 3), jnp.float32),
            pltpu.SemaphoreType.DMA,
        ],
    )
    return k(idx.astype(jnp.int32), betas, global_orient, body_pose, transl)


# speedup vs baseline: 1.7850x; 1.1422x over previous
import jax
import jax.numpy as jnp
from jax import lax
from jax.experimental import pallas as pl
from jax.experimental.pallas import tpu as pltpu
from jax.experimental.pallas import tpu_sc as plsc

_B = 4096
_NC = 2
_NS = 16
_NW = _NC * _NS
_BPW = _B // _NW
_L = 16


def _body(idx_hbm, betas_hbm, go_hbm, bp_hbm, tr_hbm,
          out_b, out_go, out_bp, out_tr,
          idx_v, b_row, go_v, bp_v, tr_v, sem):
    wid = lax.axis_index("s") * _NC + lax.axis_index("c")
    base = wid * _BPW

    pltpu.sync_copy(idx_hbm.at[pl.ds(base, _BPW)], idx_v)

    pltpu.sync_copy(betas_hbm, b_row)
    pending = []
    for j in range(_BPW):
        pending.append(
            pltpu.async_copy(b_row, out_b.at[pl.ds(base + j, 1)], sem))

    for g in range(_BPW // _L):
        v = idx_v[pl.ds(g * _L, _L)]
        for l in range(_L):
            r = g * _L + l
            s = v[l]
            pending.append(
                pltpu.async_copy(go_hbm.at[pl.ds(s, 1)], go_v.at[pl.ds(r, 1)], sem))
            pending.append(
                pltpu.async_copy(bp_hbm.at[pl.ds(s, 1)], bp_v.at[pl.ds(r, 1)], sem))
            pending.append(
                pltpu.async_copy(tr_hbm.at[pl.ds(s, 1)], tr_v.at[pl.ds(r, 1)], sem))
    for h in pending:
        h.wait()

    pltpu.sync_copy(go_v, out_go.at[pl.ds(base, _BPW)])
    pltpu.sync_copy(bp_v, out_bp.at[pl.ds(base, _BPW)])
    pltpu.sync_copy(tr_v, out_tr.at[pl.ds(base, _BPW)])


def kernel(idx, betas, global_orient, body_pose, transl):
    mesh = plsc.VectorSubcoreMesh(core_axis_name="c", subcore_axis_name="s")
    k = pl.kernel(
        _body,
        mesh=mesh,
        out_type=(
            jax.ShapeDtypeStruct((_B, 10), jnp.float32),
            jax.ShapeDtypeStruct((_B, 3), jnp.float32),
            jax.ShapeDtypeStruct((_B, 69), jnp.float32),
            jax.ShapeDtypeStruct((_B, 3), jnp.float32),
        ),
        scratch_types=[
            pltpu.VMEM((_BPW,), jnp.int32),
            pltpu.VMEM((1, 10), jnp.float32),
            pltpu.VMEM((_BPW, 3), jnp.float32),
            pltpu.VMEM((_BPW, 69), jnp.float32),
            pltpu.VMEM((_BPW, 3), jnp.float32),
            pltpu.SemaphoreType.DMA,
        ],
    )
    return k(idx.astype(jnp.int32), betas, global_orient, body_pose, transl)
